# baseline (speedup 1.0000x reference)
import jax
import jax.numpy as jnp
from jax.experimental import pallas as pl

H = 64
IN = 2
GX, GY = 4, 4
NS = 4.0
OBS = 15
N = 1024
G = GX * GY

_DN_BT = (((1,), (1,)), ((), ()))
_DN_AT = (((0,), (0,)), ((), ()))


def _fused_kernel(target_ref, others_ref, mask_ref,
                  Wih_ref, Whh_ref, b_ref, W1_ref, b1_ref,
                  W2_ref, b2_ref, Wc_ref, bc_ref, out_ref):
    f32 = jnp.float32
    Wih = Wih_ref[...]
    Whh = Whh_ref[...]
    b = b_ref[...]
    target = target_ref[...]

    xg = jax.lax.dot_general(target, Wih, _DN_BT,
                             preferred_element_type=f32)
    h = jnp.zeros((1, H), f32)
    c = jnp.zeros((1, H), f32)
    for t in range(OBS):
        gates = (xg[t:t + 1, :] + b
                 + jax.lax.dot_general(h, Whh, _DN_BT,
                                       preferred_element_type=f32))
        gi = jax.nn.sigmoid(gates[:, 0:H])
        gf = jax.nn.sigmoid(gates[:, H:2 * H])
        gg = jnp.tanh(gates[:, 2 * H:3 * H])
        go = jax.nn.sigmoid(gates[:, 3 * H:4 * H])
        c = gf * c + gi * gg
        h = go * jnp.tanh(c)

    others = others_ref[0]
    gates_o = jax.lax.dot_general(others, Wih, _DN_BT,
                                  preferred_element_type=f32) + b
    co = jax.nn.sigmoid(gates_o[:, 0:H]) * jnp.tanh(gates_o[:, 2 * H:3 * H])
    ho = jax.nn.sigmoid(gates_o[:, 3 * H:4 * H]) * jnp.tanh(co)

    cell_w = NS / GX
    cell_h = NS / GY
    px = target[OBS - 1:OBS, 0:1]
    py = target[OBS - 1:OBS, 1:2]
    rx = others[:, 0:1] - px
    ry = others[:, 1:2] - py
    within = (jnp.abs(rx) <= NS / 2) & (jnp.abs(ry) <= NS / 2)
    cx = (rx / cell_w).astype(jnp.int32) + GX // 2
    cy = (ry / cell_h).astype(jnp.int32) + GY // 2
    inb = (cx >= 0) & (cx < GX) & (cy >= 0) & (cy < GY)
    m = within & inb & (mask_ref[0, 0][:, None] != 0)
    idx = jnp.where(m, cy * GX + cx, 0)

    g_iota = jax.lax.broadcasted_iota(jnp.int32, (N, G), 1)
    onehot = ((idx == g_iota) & m).astype(f32)
    grid = jax.lax.dot_general(onehot, ho, _DN_AT,
                               preferred_element_type=f32)

    acc = b1_ref[...]
    for g in range(G):
        acc = acc + jax.lax.dot_general(
            grid[g:g + 1, :], W1_ref[:, g * H:(g + 1) * H], _DN_BT,
            preferred_element_type=f32)
    sc = jax.lax.dot_general(jnp.maximum(acc, 0.0), W2_ref[...], _DN_BT,
                             preferred_element_type=f32) + b2_ref[...]

    combined = h + sc
    out_ref[...] = (jax.lax.dot_general(combined, Wc_ref[...], _DN_BT,
                                        preferred_element_type=f32)
                    + bc_ref[...])


def kernel(observed_trajectory_target, observed_trajectory_others, neighbor_mask,
           W_ih, W_hh, b_ih, b_hh, W1, b1, W2, b2, Wc, bc):
    b_comb = (b_ih + b_hh).reshape(1, 4 * H)
    out = pl.pallas_call(
        _fused_kernel,
        grid=(1,),
        in_specs=[
            pl.BlockSpec((OBS, IN), lambda i: (0, 0)),
            pl.BlockSpec((1, N, IN), lambda i: (OBS - 1, 0, 0)),
            pl.BlockSpec((1, 1, N), lambda i: (OBS - 1, 0, 0)),
            pl.BlockSpec((4 * H, IN), lambda i: (0, 0)),
            pl.BlockSpec((4 * H, H), lambda i: (0, 0)),
            pl.BlockSpec((1, 4 * H), lambda i: (0, 0)),
            pl.BlockSpec((H, G * H), lambda i: (0, 0)),
            pl.BlockSpec((1, H), lambda i: (0, 0)),
            pl.BlockSpec((H, H), lambda i: (0, 0)),
            pl.BlockSpec((1, H), lambda i: (0, 0)),
            pl.BlockSpec((2, H), lambda i: (0, 0)),
            pl.BlockSpec((1, 2), lambda i: (0, 0)),
        ],
        out_specs=pl.BlockSpec((1, 2), lambda i: (0, 0)),
        out_shape=jax.ShapeDtypeStruct((1, 2), jnp.float32),
    )(observed_trajectory_target, observed_trajectory_others,
      neighbor_mask.reshape(OBS, 1, N),
      W_ih, W_hh, b_comb, W1, b1.reshape(1, H),
      W2, b2.reshape(1, H), Wc, bc.reshape(1, 2))
    return out


# speedup vs baseline: 1.3137x; 1.3137x over previous
import jax
import jax.numpy as jnp
from jax.experimental import pallas as pl

H = 64
IN = 2
GX, GY = 4, 4
NS = 4.0
OBS = 15
N = 1024
G = GX * GY

_DN_BT = (((1,), (1,)), ((), ()))


def _fused_kernel(A_ref, B_ref, C_ref, out_ref):
    f32 = jnp.float32
    W1 = A_ref[0:H, :]
    othersT = A_ref[H:H + IN, :]
    maskf = A_ref[H + IN:H + IN + 1, :]
    Whh = B_ref[0:4 * H, :]
    W2 = B_ref[4 * H:5 * H, :]
    Wc = B_ref[5 * H:5 * H + 2, :]
    Wih = B_ref[5 * H + 8:9 * H + 8, 0:IN]
    targetT = B_ref[9 * H + 8:9 * H + 10, 0:OBS]
    b = C_ref[0:4 * H, :]
    b1 = C_ref[4 * H:5 * H, :]
    b2 = C_ref[5 * H:6 * H, :]
    bc = C_ref[6 * H:6 * H + 2, :]

    xg = jnp.dot(Wih, targetT, preferred_element_type=f32) + b
    h = jnp.zeros((H, 1), f32)
    c = jnp.zeros((H, 1), f32)
    for t in range(OBS):
        gates = xg[:, t:t + 1] + jnp.dot(Whh, h, preferred_element_type=f32)
        gi = jax.nn.sigmoid(gates[0:H, :])
        gf = jax.nn.sigmoid(gates[H:2 * H, :])
        gg = jnp.tanh(gates[2 * H:3 * H, :])
        go = jax.nn.sigmoid(gates[3 * H:4 * H, :])
        c = gf * c + gi * gg
        h = go * jnp.tanh(c)

    gates_o = jnp.dot(Wih, othersT, preferred_element_type=f32) + b
    co = (jax.nn.sigmoid(gates_o[0:H, :])
          * jnp.tanh(gates_o[2 * H:3 * H, :]))
    hoT = jax.nn.sigmoid(gates_o[3 * H:4 * H, :]) * jnp.tanh(co)

    cell_w = NS / GX
    cell_h = NS / GY
    px = targetT[0:1, OBS - 1:OBS]
    py = targetT[1:2, OBS - 1:OBS]
    rx = othersT[0:1, :] - px
    ry = othersT[1:2, :] - py
    within = (jnp.abs(rx) <= NS / 2) & (jnp.abs(ry) <= NS / 2)
    cx = (rx / cell_w).astype(jnp.int32) + GX // 2
    cy = (ry / cell_h).astype(jnp.int32) + GY // 2
    inb = (cx >= 0) & (cx < GX) & (cy >= 0) & (cy < GY)
    m = within & inb & (maskf != 0.0)
    idx = jnp.where(m, cy * GX + cx, 0)

    g_iota = jax.lax.broadcasted_iota(jnp.int32, (G, N), 0)
    onehotT = ((idx == g_iota) & m).astype(f32)
    gridT = jax.lax.dot_general(hoT, onehotT, _DN_BT,
                                preferred_element_type=f32)

    st = jnp.concatenate([gridT[:, g:g + 1] for g in range(G)], axis=0)
    acc = jnp.dot(W1, st, preferred_element_type=f32) + b1
    sc = (jnp.dot(W2, jnp.maximum(acc, 0.0),
                  preferred_element_type=f32) + b2)

    combined = h + sc
    out_ref[...] = jnp.dot(Wc, combined, preferred_element_type=f32) + bc


def kernel(observed_trajectory_target, observed_trajectory_others, neighbor_mask,
           W_ih, W_hh, b_ih, b_hh, W1, b1, W2, b2, Wc, bc):
    othersT = observed_trajectory_others[OBS - 1].T
    maskf = neighbor_mask[OBS - 1].astype(jnp.float32)[None, :]
    A = jnp.concatenate([W1, othersT, maskf], axis=0)
    B = jnp.concatenate([
        W_hh,
        W2,
        jnp.pad(Wc, ((0, 6), (0, 0))),
        jnp.pad(W_ih, ((0, 0), (0, H - IN))),
        jnp.pad(observed_trajectory_target.T, ((0, 6), (0, H - OBS))),
    ], axis=0)
    C = jnp.concatenate([
        (b_ih + b_hh)[:, None], b1[:, None], b2[:, None], bc[:, None],
    ], axis=0)
    out = pl.pallas_call(
        _fused_kernel,
        out_shape=jax.ShapeDtypeStruct((2, 1), jnp.float32),
    )(A, B, C)
    return out.reshape(1, 2)
